# baseline (speedup 1.0000x reference)
import functools

import jax
import jax.numpy as jnp
from jax import lax
from jax.experimental import pallas as pl
from jax.experimental.pallas import tpu as pltpu
from jax.experimental.pallas import tpu_sc as plsc

_N = 10000
_E = 320000
_D = 128
_H = 128

_NC = 2
_NS = 16
_NW = _NC * _NS

_K = 48
_NCHUNK = 224
_DK = 128
_DCH = 84
_EPT = _K * _NCHUNK
_EPAD = _EPT * _NW

_NP = 10112
_RPT = _NP // _NS
_ZROWS = 128

_DEGP = 10240
_DPT = _DEGP // _NS

_ROWBLK = 1000
_GRID = _N // _ROWBLK

_mesh = plsc.VectorSubcoreMesh(core_axis_name="c", subcore_axis_name="s")


@functools.partial(
    pl.kernel,
    out_type=jax.ShapeDtypeStruct((_NC, _DEGP), jnp.float32),
    mesh=_mesh,
    scratch_types=[
        pltpu.VMEM_SHARED((_DEGP,), jnp.float32),
        pltpu.VMEM((_DK,), jnp.int32),
        pltpu.VMEM((_DK,), jnp.float32),
        pltpu.VMEM((_DPT,), jnp.float32),
    ],
)
def _sc_degree(c_hbm, ew_hbm, out_hbm, deg_sh, c_v, ew_v, zbuf):
    cid = lax.axis_index("c")
    sid = lax.axis_index("s")
    wid = sid * _NC + cid

    for i in range(_DPT // 16):
        zbuf[pl.ds(i * 16, 16)] = jnp.zeros((16,), jnp.float32)
    pltpu.sync_copy(zbuf, deg_sh.at[pl.ds(sid * _DPT, _DPT)])
    plsc.subcore_barrier()

    base = wid * _EPT

    def chunk(i, carry):
        off = base + i * _DK
        pltpu.sync_copy(c_hbm.at[pl.ds(off, _DK)], c_v)
        pltpu.sync_copy(ew_hbm.at[pl.ds(off, _DK)], ew_v)
        pltpu.sync_copy(ew_v, deg_sh.at[c_v], add=True)
        return carry

    lax.fori_loop(0, _DCH, chunk, 0)
    plsc.subcore_barrier()
    pltpu.sync_copy(deg_sh.at[pl.ds(sid * _DPT, _DPT)],
                    out_hbm.at[cid, pl.ds(sid * _DPT, _DPT)])


@functools.partial(
    pl.kernel,
    out_type=jax.ShapeDtypeStruct((_NC, _NP, _H), jnp.float32),
    mesh=_mesh,
    compiler_params=pltpu.CompilerParams(use_tc_tiling_on_sc=False,
                                        needs_layout_passes=False),
    scratch_types=[
        pltpu.VMEM_SHARED((_NP, _H), jnp.float32),
        pltpu.VMEM_SHARED((_NP, _H // 2), jnp.int32),
        pltpu.VMEM((_K, _H // 2), jnp.int32),
        pltpu.VMEM((_K, _H), jnp.float32),
        pltpu.VMEM((4 * _K,), jnp.int32),
        pltpu.VMEM((4 * _K,), jnp.int32),
        pltpu.VMEM((_K,), jnp.int32),
        pltpu.VMEM((_K,), jnp.int32),
        pltpu.SemaphoreType.DMA,
        pltpu.SemaphoreType.DMA,
        pltpu.SemaphoreType.DMA,
        pltpu.SemaphoreType.DMA,
        pltpu.SemaphoreType.DMA,
    ],
)
def _sc_message(gp_hbm, rw_hbm, c_hbm, out_hbm,
                acc, g_sh, gb, stage, rw0, rw1, cb0, cb1,
                gsem, rwsem0, rwsem1, csem0, csem1):
    cid = lax.axis_index("c")
    sid = lax.axis_index("s")
    wid = sid * _NC + cid
    rwb = (rw0, rw1)
    cb = (cb0, cb1)
    rwsem = (rwsem0, rwsem1)
    csem = (csem0, csem1)

    def zrow(i, carry):
        for q in range(_H // 16):
            stage[i, pl.ds(q * 16, 16)] = jnp.zeros((16,), jnp.float32)
        return carry

    lax.fori_loop(0, _K, zrow, 0)
    pltpu.sync_copy(gp_hbm.at[pl.ds(sid * _RPT, _RPT)],
                    g_sh.at[pl.ds(sid * _RPT, _RPT)])
    for t in range(13):
        pltpu.sync_copy(stage,
                        acc.at[pl.ds(sid * _RPT + t * _K, _K)])
    pltpu.sync_copy(stage.at[pl.ds(0, _RPT - 13 * _K)],
                    acc.at[pl.ds(sid * _RPT + 13 * _K, _RPT - 13 * _K)])
    plsc.subcore_barrier()

    nblk = _NCHUNK // 2
    rwbase = wid * nblk * 4 * _K
    cbase = wid * _EPT

    def rw_desc(blk, slot):
        return pltpu.make_async_copy(
            rw_hbm.at[pl.ds(rwbase + blk * 4 * _K, 4 * _K)], rwb[slot],
            rwsem[slot])

    def c_desc(i, slot):
        return pltpu.make_async_copy(
            c_hbm.at[pl.ds(cbase + i * _K, _K)], cb[slot], csem[slot])

    for d in (rw_desc(0, 0), c_desc(0, 0)):
        d.start()
    for d in (rw_desc(0, 0), c_desc(0, 0)):
        d.wait()

    def blk2(bo, carry):
        for bslot in range(2):
            blk = bo * 2 + bslot
            for sub in range(2):
                i = blk * 2 + sub
                pltpu.async_copy(
                    g_sh.at[rwb[bslot].at[pl.ds(sub * _K, _K)]], gb,
                    gsem).wait()

                if sub == 0:
                    @pl.when(blk + 1 < nblk)
                    def _next_rw():
                        rw_desc(blk + 1, 1 - bslot).start()

                @pl.when(i + 1 < _NCHUNK)
                def _next_c():
                    c_desc(i + 1, 1 - sub).start()

                def scale16(jo, inner):
                    j0 = jo * 16
                    wv = plsc.bitcast(
                        rwb[bslot][pl.ds(2 * _K + sub * _K + j0, 16)],
                        jnp.float32)
                    for jj in range(16):
                        w = jnp.full((16,), wv[jj], jnp.float32)
                        for q in range(_H // 32):
                            u = gb[j0 + jj, pl.ds(q * 16, 16)]
                            lo = plsc.bitcast(u << 16, jnp.float32)
                            hi = plsc.bitcast(
                                u & jnp.int32(-65536), jnp.float32)
                            stage[j0 + jj, pl.ds(q * 32, 16)] = lo * w
                            stage[j0 + jj, pl.ds(q * 32 + 16, 16)] = hi * w
                    return inner

                lax.fori_loop(0, _K // 16, scale16, 0)

                pltpu.sync_copy(stage, acc.at[cb[sub]], add=True)

                @pl.when(i + 1 < _NCHUNK)
                def _land_c():
                    c_desc(i + 1, 1 - sub).wait()

                if sub == 1:
                    @pl.when(blk + 1 < nblk)
                    def _land_rw():
                        rw_desc(blk + 1, 1 - bslot).wait()
        return carry

    lax.fori_loop(0, nblk // 2, blk2, 0)
    plsc.subcore_barrier()
    for t in range(13):
        o = sid * _RPT + t * _K
        pltpu.sync_copy(acc.at[pl.ds(o, _K)], out_hbm.at[cid, pl.ds(o, _K)])
    o = sid * _RPT + 13 * _K
    pltpu.sync_copy(acc.at[pl.ds(o, _RPT - 13 * _K)],
                    out_hbm.at[cid, pl.ds(o, _RPT - 13 * _K)])


def _tc1_body(d0_ref, d1_ref, x_ref, w1_ref, dis_ref, g1_ref):
    deg = d0_ref[...] + d1_ref[...] + 1.0
    dis = jnp.where(deg > 0.0, lax.rsqrt(deg), 0.0)
    dis_ref[...] = dis
    h = jnp.dot(x_ref[...], w1_ref[...], preferred_element_type=jnp.float32)
    g1_ref[...] = h * dis


def _tc2_body(a0_ref, a1_ref, g1_ref, dis_ref, b1_ref, w2_ref,
              h1_ref, g2_ref):
    dis = dis_ref[...]
    pre = (a0_ref[...] + a1_ref[...] + g1_ref[...]) * dis + b1_ref[...]
    h1 = jnp.maximum(pre, 0.0)
    h1_ref[...] = h1
    g2_ref[...] = jnp.dot(h1, w2_ref[...],
                          preferred_element_type=jnp.float32) * dis


def _tc3_body(a0_ref, a1_ref, g2_ref, dis_ref, b2_ref, h2_ref):
    pre = ((a0_ref[...] + a1_ref[...] + g2_ref[...]) * dis_ref[...]
           + b2_ref[...])
    h2_ref[...] = jnp.maximum(pre, 0.0)


def _row_blk(shape_cols):
    return pl.BlockSpec((_ROWBLK, shape_cols), lambda i: (i, 0))


def _full_blk(rows, cols):
    return pl.BlockSpec((rows, cols), lambda i: (0, 0))


_tc1 = pl.pallas_call(
    _tc1_body,
    grid=(_GRID,),
    in_specs=[
        _row_blk(1), _row_blk(1), _row_blk(_D), _full_blk(_D, _H),
    ],
    out_specs=[_row_blk(1), _row_blk(_H)],
    out_shape=[
        jax.ShapeDtypeStruct((_N, 1), jnp.float32),
        jax.ShapeDtypeStruct((_N, _H), jnp.float32),
    ],
)

_tc2 = pl.pallas_call(
    _tc2_body,
    grid=(_GRID,),
    in_specs=[
        _row_blk(_H), _row_blk(_H), _row_blk(_H), _row_blk(1),
        _full_blk(1, _H), _full_blk(_H, _H),
    ],
    out_specs=[_row_blk(_H), _row_blk(_H)],
    out_shape=[
        jax.ShapeDtypeStruct((_N, _H), jnp.float32),
        jax.ShapeDtypeStruct((_N, _H), jnp.float32),
    ],
)

_tc3 = pl.pallas_call(
    _tc3_body,
    grid=(_GRID,),
    in_specs=[
        _row_blk(_H), _row_blk(_H), _row_blk(_H), _row_blk(1),
        _full_blk(1, _H),
    ],
    out_specs=_row_blk(_H),
    out_shape=jax.ShapeDtypeStruct((_N, _H), jnp.float32),
)


def _pack_bf16(g):
    ge = g.astype(jnp.bfloat16).reshape(_N, _H // 32, 2, 16)
    gt = ge.transpose(0, 1, 3, 2)
    gi = lax.bitcast_convert_type(gt, jnp.int32).reshape(_N, _H // 2)
    return jnp.concatenate(
        [gi, jnp.zeros((_NP - _N, _H // 2), jnp.int32)], axis=0)


@jax.jit
def kernel(x, edge_index, edge_weights, W1, b1, W2, b2):
    row = edge_index[0]
    col = edge_index[1]
    pad = _EPAD - _E
    row_p = jnp.concatenate([row, jnp.zeros((pad,), jnp.int32)])
    col_p = jnp.concatenate([col, jnp.zeros((pad,), jnp.int32)])
    ew_p = jnp.concatenate([edge_weights, jnp.zeros((pad,), jnp.float32)])

    deg_parts = _sc_degree(col_p, ew_p)
    d0 = deg_parts[0, :_N].reshape(_N, 1)
    d1 = deg_parts[1, :_N].reshape(_N, 1)

    dis, g1 = _tc1(d0, d1, x, W1)

    ew_i = lax.bitcast_convert_type(ew_p, jnp.int32)
    rw = jnp.stack([row_p.reshape(-1, 2 * _K),
                    ew_i.reshape(-1, 2 * _K)], axis=1).reshape(-1)

    acc1 = _sc_message(_pack_bf16(g1), rw, col_p)
    h1, g2 = _tc2(acc1[0, :_N], acc1[1, :_N], g1, dis,
                  b1.reshape(1, _H), W2)

    acc2 = _sc_message(_pack_bf16(g2), rw, col_p)
    h2 = _tc3(acc2[0, :_N], acc2[1, :_N], g2, dis, b2.reshape(1, _H))

    return jnp.concatenate([h1, h2], axis=-1)


# speedup vs baseline: 1.1012x; 1.1012x over previous
import functools

import jax
import jax.numpy as jnp
from jax import lax
from jax.experimental import pallas as pl
from jax.experimental.pallas import tpu as pltpu
from jax.experimental.pallas import tpu_sc as plsc

_N = 10000
_E = 320000
_D = 128
_H = 128

_NC = 2
_NS = 16
_NW = _NC * _NS

_K = 48
_NCHUNK = 224
_DK = 128
_DCH = 84
_EPT = _K * _NCHUNK
_EPAD = _EPT * _NW

_NP = 10112
_RPT = _NP // _NS
_ZROWS = 128

_DEGP = 10240
_DPT = _DEGP // _NS

_ROWBLK = 1000
_GRID = _N // _ROWBLK

_mesh = plsc.VectorSubcoreMesh(core_axis_name="c", subcore_axis_name="s")


@functools.partial(
    pl.kernel,
    out_type=jax.ShapeDtypeStruct((_NC, _DEGP), jnp.float32),
    mesh=_mesh,
    scratch_types=[
        pltpu.VMEM_SHARED((_DEGP,), jnp.float32),
        pltpu.VMEM((_DK,), jnp.int32),
        pltpu.VMEM((_DK,), jnp.float32),
        pltpu.VMEM((_DPT,), jnp.float32),
    ],
)
def _sc_degree(c_hbm, ew_hbm, out_hbm, deg_sh, c_v, ew_v, zbuf):
    cid = lax.axis_index("c")
    sid = lax.axis_index("s")
    wid = sid * _NC + cid

    for i in range(_DPT // 16):
        zbuf[pl.ds(i * 16, 16)] = jnp.zeros((16,), jnp.float32)
    pltpu.sync_copy(zbuf, deg_sh.at[pl.ds(sid * _DPT, _DPT)])
    plsc.subcore_barrier()

    base = wid * _EPT

    def chunk(i, carry):
        off = base + i * _DK
        pltpu.sync_copy(c_hbm.at[pl.ds(off, _DK)], c_v)
        pltpu.sync_copy(ew_hbm.at[pl.ds(off, _DK)], ew_v)
        pltpu.sync_copy(ew_v, deg_sh.at[c_v], add=True)
        return carry

    lax.fori_loop(0, _DCH, chunk, 0)
    plsc.subcore_barrier()
    pltpu.sync_copy(deg_sh.at[pl.ds(sid * _DPT, _DPT)],
                    out_hbm.at[cid, pl.ds(sid * _DPT, _DPT)])


@functools.partial(
    pl.kernel,
    out_type=jax.ShapeDtypeStruct((_NC, _NP, _H), jnp.float32),
    mesh=_mesh,
    compiler_params=pltpu.CompilerParams(use_tc_tiling_on_sc=False,
                                        needs_layout_passes=False),
    scratch_types=[
        pltpu.VMEM_SHARED((_NP, _H), jnp.float32),
        pltpu.VMEM_SHARED((_NP, _H // 2), jnp.int32),
        pltpu.VMEM((_K, _H // 2), jnp.int32),
        pltpu.VMEM((_K, _H), jnp.float32),
        pltpu.VMEM((4 * _K,), jnp.int32),
        pltpu.VMEM((4 * _K,), jnp.int32),
        pltpu.VMEM((_K,), jnp.int32),
        pltpu.VMEM((_K,), jnp.int32),
        pltpu.SemaphoreType.DMA,
        pltpu.SemaphoreType.DMA,
        pltpu.SemaphoreType.DMA,
        pltpu.SemaphoreType.DMA,
        pltpu.SemaphoreType.DMA,
        pltpu.SemaphoreType.DMA,
    ],
)
def _sc_message(gp_hbm, rw_hbm, c_hbm, out_hbm,
                acc, g_sh, gb, stage, rw0, rw1, cb0, cb1,
                gsem, ssem, rwsem0, rwsem1, csem0, csem1):
    cid = lax.axis_index("c")
    sid = lax.axis_index("s")
    wid = sid * _NC + cid
    rwb = (rw0, rw1)
    cb = (cb0, cb1)
    rwsem = (rwsem0, rwsem1)
    csem = (csem0, csem1)

    def zrow(i, carry):
        for q in range(_H // 16):
            stage[i, pl.ds(q * 16, 16)] = jnp.zeros((16,), jnp.float32)
        return carry

    lax.fori_loop(0, _K, zrow, 0)
    pltpu.sync_copy(gp_hbm.at[pl.ds(sid * _RPT, _RPT)],
                    g_sh.at[pl.ds(sid * _RPT, _RPT)])
    for t in range(13):
        pltpu.sync_copy(stage,
                        acc.at[pl.ds(sid * _RPT + t * _K, _K)])
    pltpu.sync_copy(stage.at[pl.ds(0, _RPT - 13 * _K)],
                    acc.at[pl.ds(sid * _RPT + 13 * _K, _RPT - 13 * _K)])
    plsc.subcore_barrier()

    nblk = _NCHUNK // 2
    rwbase = wid * nblk * 4 * _K
    cbase = wid * _EPT

    def rw_desc(blk, slot):
        return pltpu.make_async_copy(
            rw_hbm.at[pl.ds(rwbase + blk * 4 * _K, 4 * _K)], rwb[slot],
            rwsem[slot])

    def c_desc(i, slot):
        return pltpu.make_async_copy(
            c_hbm.at[pl.ds(cbase + i * _K, _K)], cb[slot], csem[slot])

    for d in (rw_desc(0, 0), c_desc(0, 0)):
        d.start()
    for d in (rw_desc(0, 0), c_desc(0, 0)):
        d.wait()

    def blk2(bo, carry):
        for bslot in range(2):
            blk = bo * 2 + bslot
            for sub in range(2):
                i = blk * 2 + sub
                pltpu.async_copy(
                    g_sh.at[rwb[bslot].at[pl.ds(sub * _K, _K)]], gb,
                    gsem).wait()

                @pl.when(i > 0)
                def _drain_scatter():
                    pltpu.make_async_copy(
                        stage, acc.at[cb[1 - sub]], ssem).wait()

                if sub == 0:
                    @pl.when(blk + 1 < nblk)
                    def _next_rw():
                        rw_desc(blk + 1, 1 - bslot).start()

                @pl.when(i + 1 < _NCHUNK)
                def _next_c():
                    c_desc(i + 1, 1 - sub).start()

                def scale16(jo, inner):
                    j0 = jo * 16
                    wv = plsc.bitcast(
                        rwb[bslot][pl.ds(2 * _K + sub * _K + j0, 16)],
                        jnp.float32)
                    for jj in range(16):
                        w = lax.gather(
                            wv, jnp.full((16, 1), jj, jnp.int32),
                            lax.GatherDimensionNumbers(
                                offset_dims=(), collapsed_slice_dims=(0,),
                                start_index_map=(0,)),
                            (1,),
                            mode=lax.GatherScatterMode.PROMISE_IN_BOUNDS)
                        for q in range(_H // 32):
                            u = gb[j0 + jj, pl.ds(q * 16, 16)]
                            lo = plsc.bitcast(u << 16, jnp.float32)
                            hi = plsc.bitcast(
                                u & jnp.int32(-65536), jnp.float32)
                            stage[j0 + jj, pl.ds(q * 32, 16)] = lo * w
                            stage[j0 + jj, pl.ds(q * 32 + 16, 16)] = hi * w
                    return inner

                lax.fori_loop(0, _K // 16, scale16, 0)

                pltpu.async_copy(stage, acc.at[cb[sub]], ssem, add=True)

                @pl.when(i + 1 < _NCHUNK)
                def _land_c():
                    c_desc(i + 1, 1 - sub).wait()

                if sub == 1:
                    @pl.when(blk + 1 < nblk)
                    def _land_rw():
                        rw_desc(blk + 1, 1 - bslot).wait()
        return carry

    lax.fori_loop(0, nblk // 2, blk2, 0)
    pltpu.make_async_copy(stage, acc.at[cb[1]], ssem).wait()
    plsc.subcore_barrier()
    for t in range(13):
        o = sid * _RPT + t * _K
        pltpu.sync_copy(acc.at[pl.ds(o, _K)], out_hbm.at[cid, pl.ds(o, _K)])
    o = sid * _RPT + 13 * _K
    pltpu.sync_copy(acc.at[pl.ds(o, _RPT - 13 * _K)],
                    out_hbm.at[cid, pl.ds(o, _RPT - 13 * _K)])


def _tc1_body(d0_ref, d1_ref, x_ref, w1_ref, dis_ref, g1_ref):
    deg = d0_ref[...] + d1_ref[...] + 1.0
    dis = jnp.where(deg > 0.0, lax.rsqrt(deg), 0.0)
    dis_ref[...] = dis
    h = jnp.dot(x_ref[...], w1_ref[...], preferred_element_type=jnp.float32)
    g1_ref[...] = h * dis


def _tc2_body(a0_ref, a1_ref, g1_ref, dis_ref, b1_ref, w2_ref,
              h1_ref, g2_ref):
    dis = dis_ref[...]
    pre = (a0_ref[...] + a1_ref[...] + g1_ref[...]) * dis + b1_ref[...]
    h1 = jnp.maximum(pre, 0.0)
    h1_ref[...] = h1
    g2_ref[...] = jnp.dot(h1, w2_ref[...],
                          preferred_element_type=jnp.float32) * dis


def _tc3_body(a0_ref, a1_ref, g2_ref, dis_ref, b2_ref, h2_ref):
    pre = ((a0_ref[...] + a1_ref[...] + g2_ref[...]) * dis_ref[...]
           + b2_ref[...])
    h2_ref[...] = jnp.maximum(pre, 0.0)


def _row_blk(shape_cols):
    return pl.BlockSpec((_ROWBLK, shape_cols), lambda i: (i, 0))


def _full_blk(rows, cols):
    return pl.BlockSpec((rows, cols), lambda i: (0, 0))


_tc1 = pl.pallas_call(
    _tc1_body,
    grid=(_GRID,),
    in_specs=[
        _row_blk(1), _row_blk(1), _row_blk(_D), _full_blk(_D, _H),
    ],
    out_specs=[_row_blk(1), _row_blk(_H)],
    out_shape=[
        jax.ShapeDtypeStruct((_N, 1), jnp.float32),
        jax.ShapeDtypeStruct((_N, _H), jnp.float32),
    ],
)

_tc2 = pl.pallas_call(
    _tc2_body,
    grid=(_GRID,),
    in_specs=[
        _row_blk(_H), _row_blk(_H), _row_blk(_H), _row_blk(1),
        _full_blk(1, _H), _full_blk(_H, _H),
    ],
    out_specs=[_row_blk(_H), _row_blk(_H)],
    out_shape=[
        jax.ShapeDtypeStruct((_N, _H), jnp.float32),
        jax.ShapeDtypeStruct((_N, _H), jnp.float32),
    ],
)

_tc3 = pl.pallas_call(
    _tc3_body,
    grid=(_GRID,),
    in_specs=[
        _row_blk(_H), _row_blk(_H), _row_blk(_H), _row_blk(1),
        _full_blk(1, _H),
    ],
    out_specs=_row_blk(_H),
    out_shape=jax.ShapeDtypeStruct((_N, _H), jnp.float32),
)


def _pack_bf16(g):
    ge = g.astype(jnp.bfloat16).reshape(_N, _H // 32, 2, 16)
    gt = ge.transpose(0, 1, 3, 2)
    gi = lax.bitcast_convert_type(gt, jnp.int32).reshape(_N, _H // 2)
    return jnp.concatenate(
        [gi, jnp.zeros((_NP - _N, _H // 2), jnp.int32)], axis=0)


@jax.jit
def kernel(x, edge_index, edge_weights, W1, b1, W2, b2):
    row = edge_index[0]
    col = edge_index[1]
    pad = _EPAD - _E
    row_p = jnp.concatenate([row, jnp.zeros((pad,), jnp.int32)])
    col_p = jnp.concatenate([col, jnp.zeros((pad,), jnp.int32)])
    ew_p = jnp.concatenate([edge_weights, jnp.zeros((pad,), jnp.float32)])

    deg_parts = _sc_degree(col_p, ew_p)
    d0 = deg_parts[0, :_N].reshape(_N, 1)
    d1 = deg_parts[1, :_N].reshape(_N, 1)

    dis, g1 = _tc1(d0, d1, x, W1)

    ew_i = lax.bitcast_convert_type(ew_p, jnp.int32)
    rw = jnp.stack([row_p.reshape(-1, 2 * _K),
                    ew_i.reshape(-1, 2 * _K)], axis=1).reshape(-1)

    acc1 = _sc_message(_pack_bf16(g1), rw, col_p)
    h1, g2 = _tc2(acc1[0, :_N], acc1[1, :_N], g1, dis,
                  b1.reshape(1, _H), W2)

    acc2 = _sc_message(_pack_bf16(g2), rw, col_p)
    h2 = _tc3(acc2[0, :_N], acc2[1, :_N], g2, dis, b2.reshape(1, _H))

    return jnp.concatenate([h1, h2], axis=-1)


# speedup vs baseline: 1.1434x; 1.0383x over previous
import functools

import jax
import jax.numpy as jnp
from jax import lax
from jax.experimental import pallas as pl
from jax.experimental.pallas import tpu as pltpu
from jax.experimental.pallas import tpu_sc as plsc

_N = 10000
_E = 320000
_D = 128
_H = 128

_NC = 2
_NS = 16
_NW = _NC * _NS

_K = 48
_NCHUNK = 224
_DK = 128
_DCH = 84
_EPT = _K * _NCHUNK
_EPAD = _EPT * _NW

_NP = 10112
_RPT = _NP // _NS
_ZROWS = 128

_DEGP = 10240
_DPT = _DEGP // _NS

_ROWBLK = 1000
_GRID = _N // _ROWBLK

_mesh = plsc.VectorSubcoreMesh(core_axis_name="c", subcore_axis_name="s")


@functools.partial(
    pl.kernel,
    out_type=jax.ShapeDtypeStruct((_NC, _DEGP), jnp.float32),
    mesh=_mesh,
    scratch_types=[
        pltpu.VMEM_SHARED((_DEGP,), jnp.float32),
        pltpu.VMEM((_DK,), jnp.int32),
        pltpu.VMEM((_DK,), jnp.int32),
        pltpu.VMEM((_DK,), jnp.float32),
        pltpu.VMEM((_DK,), jnp.float32),
        pltpu.VMEM((_DPT,), jnp.float32),
        pltpu.SemaphoreType.DMA,
        pltpu.SemaphoreType.DMA,
        pltpu.SemaphoreType.DMA,
    ],
)
def _sc_degree(c_hbm, ew_hbm, out_hbm, deg_sh, c0, c1, w0, w1, zbuf,
               isem0, isem1, ssem):
    cid = lax.axis_index("c")
    sid = lax.axis_index("s")
    wid = sid * _NC + cid
    c_v = (c0, c1)
    ew_v = (w0, w1)
    isem = (isem0, isem1)

    for i in range(_DPT // 16):
        zbuf[pl.ds(i * 16, 16)] = jnp.zeros((16,), jnp.float32)
    pltpu.sync_copy(zbuf, deg_sh.at[pl.ds(sid * _DPT, _DPT)])
    plsc.subcore_barrier()

    base = wid * _EPT

    def idx_descs(i, b):
        off = base + i * _DK
        return (
            pltpu.make_async_copy(c_hbm.at[pl.ds(off, _DK)], c_v[b],
                                  isem[b]),
            pltpu.make_async_copy(ew_hbm.at[pl.ds(off, _DK)], ew_v[b],
                                  isem[b]),
        )

    for d in idx_descs(0, 0):
        d.start()
    for d in idx_descs(0, 0):
        d.wait()

    def chunk2(io, carry):
        for b in range(2):
            i = io * 2 + b
            nb = 1 - b

            @pl.when(i + 1 < _DCH)
            def _next_idx():
                for d in idx_descs(i + 1, nb):
                    d.start()

            @pl.when(i > 0)
            def _drain():
                pltpu.make_async_copy(ew_v[nb], deg_sh.at[c_v[nb]],
                                      ssem).wait()

            pltpu.async_copy(ew_v[b], deg_sh.at[c_v[b]], ssem, add=True)

            @pl.when(i + 1 < _DCH)
            def _land_idx():
                for d in idx_descs(i + 1, nb):
                    d.wait()
        return carry

    lax.fori_loop(0, _DCH // 2, chunk2, 0)
    pltpu.make_async_copy(ew_v[1], deg_sh.at[c_v[1]], ssem).wait()
    plsc.subcore_barrier()
    pltpu.sync_copy(deg_sh.at[pl.ds(sid * _DPT, _DPT)],
                    out_hbm.at[cid, pl.ds(sid * _DPT, _DPT)])


@functools.partial(
    pl.kernel,
    out_type=jax.ShapeDtypeStruct((_NC, _NP, _H), jnp.float32),
    mesh=_mesh,
    compiler_params=pltpu.CompilerParams(use_tc_tiling_on_sc=False,
                                        needs_layout_passes=False),
    scratch_types=[
        pltpu.VMEM_SHARED((_NP, _H), jnp.float32),
        pltpu.VMEM_SHARED((_NP, _H // 2), jnp.int32),
        pltpu.VMEM((_K, _H // 2), jnp.int32),
        pltpu.VMEM((_K, _H), jnp.float32),
        pltpu.VMEM((4 * _K,), jnp.int32),
        pltpu.VMEM((4 * _K,), jnp.int32),
        pltpu.VMEM((_K,), jnp.int32),
        pltpu.VMEM((_K,), jnp.int32),
        pltpu.SemaphoreType.DMA,
        pltpu.SemaphoreType.DMA,
        pltpu.SemaphoreType.DMA,
        pltpu.SemaphoreType.DMA,
        pltpu.SemaphoreType.DMA,
        pltpu.SemaphoreType.DMA,
    ],
)
def _sc_message(gp_hbm, rw_hbm, c_hbm, out_hbm,
                acc, g_sh, gb, stage, rw0, rw1, cb0, cb1,
                gsem, ssem, rwsem0, rwsem1, csem0, csem1):
    cid = lax.axis_index("c")
    sid = lax.axis_index("s")
    wid = sid * _NC + cid
    rwb = (rw0, rw1)
    cb = (cb0, cb1)
    rwsem = (rwsem0, rwsem1)
    csem = (csem0, csem1)

    def zrow(i, carry):
        for q in range(_H // 16):
            stage[i, pl.ds(q * 16, 16)] = jnp.zeros((16,), jnp.float32)
        return carry

    lax.fori_loop(0, _K, zrow, 0)
    pltpu.sync_copy(gp_hbm.at[pl.ds(sid * _RPT, _RPT)],
                    g_sh.at[pl.ds(sid * _RPT, _RPT)])
    for t in range(13):
        pltpu.sync_copy(stage,
                        acc.at[pl.ds(sid * _RPT + t * _K, _K)])
    pltpu.sync_copy(stage.at[pl.ds(0, _RPT - 13 * _K)],
                    acc.at[pl.ds(sid * _RPT + 13 * _K, _RPT - 13 * _K)])
    plsc.subcore_barrier()

    nblk = _NCHUNK // 2
    rwbase = wid * nblk * 4 * _K
    cbase = wid * _EPT

    def rw_desc(blk, slot):
        return pltpu.make_async_copy(
            rw_hbm.at[pl.ds(rwbase + blk * 4 * _K, 4 * _K)], rwb[slot],
            rwsem[slot])

    def c_desc(i, slot):
        return pltpu.make_async_copy(
            c_hbm.at[pl.ds(cbase + i * _K, _K)], cb[slot], csem[slot])

    for d in (rw_desc(0, 0), c_desc(0, 0)):
        d.start()
    for d in (rw_desc(0, 0), c_desc(0, 0)):
        d.wait()
    pltpu.async_copy(g_sh.at[rw0.at[pl.ds(0, _K)]], gb, gsem)

    def blk2(bo, carry):
        for bslot in range(2):
            blk = bo * 2 + bslot
            for sub in range(2):
                i = blk * 2 + sub
                pltpu.make_async_copy(
                    g_sh.at[rwb[bslot].at[pl.ds(sub * _K, _K)]], gb,
                    gsem).wait()

                @pl.when(i > 0)
                def _drain_scatter():
                    pltpu.make_async_copy(
                        stage, acc.at[cb[1 - sub]], ssem).wait()

                if sub == 0:
                    @pl.when(blk + 1 < nblk)
                    def _next_rw():
                        rw_desc(blk + 1, 1 - bslot).start()

                @pl.when(i + 1 < _NCHUNK)
                def _next_c():
                    c_desc(i + 1, 1 - sub).start()

                def scale16(jo, inner):
                    j0 = jo * 16
                    wv = plsc.bitcast(
                        rwb[bslot][pl.ds(2 * _K + sub * _K + j0, 16)],
                        jnp.float32)
                    for jj in range(16):
                        w = lax.gather(
                            wv, jnp.full((16, 1), jj, jnp.int32),
                            lax.GatherDimensionNumbers(
                                offset_dims=(), collapsed_slice_dims=(0,),
                                start_index_map=(0,)),
                            (1,),
                            mode=lax.GatherScatterMode.PROMISE_IN_BOUNDS)
                        for q in range(_H // 32):
                            u = gb[j0 + jj, pl.ds(q * 16, 16)]
                            lo = plsc.bitcast(u << 16, jnp.float32)
                            hi = plsc.bitcast(
                                u & jnp.int32(-65536), jnp.float32)
                            stage[j0 + jj, pl.ds(q * 32, 16)] = lo * w
                            stage[j0 + jj, pl.ds(q * 32 + 16, 16)] = hi * w
                    return inner

                lax.fori_loop(0, _K // 16, scale16, 0)

                if sub == 0:
                    pltpu.async_copy(
                        g_sh.at[rwb[bslot].at[pl.ds(_K, _K)]], gb, gsem)
                else:
                    @pl.when(blk + 1 < nblk)
                    def _next_gather():
                        rw_desc(blk + 1, 1 - bslot).wait()
                        pltpu.async_copy(
                            g_sh.at[rwb[1 - bslot].at[pl.ds(0, _K)]], gb,
                            gsem)

                pltpu.async_copy(stage, acc.at[cb[sub]], ssem, add=True)

                @pl.when(i + 1 < _NCHUNK)
                def _land_c():
                    c_desc(i + 1, 1 - sub).wait()
        return carry

    lax.fori_loop(0, nblk // 2, blk2, 0)
    pltpu.make_async_copy(stage, acc.at[cb[1]], ssem).wait()
    plsc.subcore_barrier()
    for t in range(13):
        o = sid * _RPT + t * _K
        pltpu.sync_copy(acc.at[pl.ds(o, _K)], out_hbm.at[cid, pl.ds(o, _K)])
    o = sid * _RPT + 13 * _K
    pltpu.sync_copy(acc.at[pl.ds(o, _RPT - 13 * _K)],
                    out_hbm.at[cid, pl.ds(o, _RPT - 13 * _K)])


def _tc1_body(d0_ref, d1_ref, x_ref, w1_ref, dis_ref, g1_ref):
    deg = d0_ref[...] + d1_ref[...] + 1.0
    dis = jnp.where(deg > 0.0, lax.rsqrt(deg), 0.0)
    dis_ref[...] = dis
    h = jnp.dot(x_ref[...], w1_ref[...], preferred_element_type=jnp.float32)
    g1_ref[...] = h * dis


def _tc2_body(a0_ref, a1_ref, g1_ref, dis_ref, b1_ref, w2_ref,
              h1_ref, g2_ref):
    dis = dis_ref[...]
    pre = (a0_ref[...] + a1_ref[...] + g1_ref[...]) * dis + b1_ref[...]
    h1 = jnp.maximum(pre, 0.0)
    h1_ref[...] = h1
    g2_ref[...] = jnp.dot(h1, w2_ref[...],
                          preferred_element_type=jnp.float32) * dis


def _tc3_body(a0_ref, a1_ref, g2_ref, dis_ref, b2_ref, h2_ref):
    pre = ((a0_ref[...] + a1_ref[...] + g2_ref[...]) * dis_ref[...]
           + b2_ref[...])
    h2_ref[...] = jnp.maximum(pre, 0.0)


def _row_blk(shape_cols):
    return pl.BlockSpec((_ROWBLK, shape_cols), lambda i: (i, 0))


def _full_blk(rows, cols):
    return pl.BlockSpec((rows, cols), lambda i: (0, 0))


_tc1 = pl.pallas_call(
    _tc1_body,
    grid=(_GRID,),
    in_specs=[
        _row_blk(1), _row_blk(1), _row_blk(_D), _full_blk(_D, _H),
    ],
    out_specs=[_row_blk(1), _row_blk(_H)],
    out_shape=[
        jax.ShapeDtypeStruct((_N, 1), jnp.float32),
        jax.ShapeDtypeStruct((_N, _H), jnp.float32),
    ],
)

_tc2 = pl.pallas_call(
    _tc2_body,
    grid=(_GRID,),
    in_specs=[
        _row_blk(_H), _row_blk(_H), _row_blk(_H), _row_blk(1),
        _full_blk(1, _H), _full_blk(_H, _H),
    ],
    out_specs=[_row_blk(_H), _row_blk(_H)],
    out_shape=[
        jax.ShapeDtypeStruct((_N, _H), jnp.float32),
        jax.ShapeDtypeStruct((_N, _H), jnp.float32),
    ],
)

_tc3 = pl.pallas_call(
    _tc3_body,
    grid=(_GRID,),
    in_specs=[
        _row_blk(_H), _row_blk(_H), _row_blk(_H), _row_blk(1),
        _full_blk(1, _H),
    ],
    out_specs=_row_blk(_H),
    out_shape=jax.ShapeDtypeStruct((_N, _H), jnp.float32),
)


def _pack_bf16(g):
    ge = g.astype(jnp.bfloat16).reshape(_N, _H // 32, 2, 16)
    gt = ge.transpose(0, 1, 3, 2)
    gi = lax.bitcast_convert_type(gt, jnp.int32).reshape(_N, _H // 2)
    return jnp.concatenate(
        [gi, jnp.zeros((_NP - _N, _H // 2), jnp.int32)], axis=0)


@jax.jit
def kernel(x, edge_index, edge_weights, W1, b1, W2, b2):
    row = edge_index[0]
    col = edge_index[1]
    pad = _EPAD - _E
    row_p = jnp.concatenate([row, jnp.zeros((pad,), jnp.int32)])
    col_p = jnp.concatenate([col, jnp.zeros((pad,), jnp.int32)])
    ew_p = jnp.concatenate([edge_weights, jnp.zeros((pad,), jnp.float32)])

    deg_parts = _sc_degree(col_p, ew_p)
    d0 = deg_parts[0, :_N].reshape(_N, 1)
    d1 = deg_parts[1, :_N].reshape(_N, 1)

    dis, g1 = _tc1(d0, d1, x, W1)

    ew_i = lax.bitcast_convert_type(ew_p, jnp.int32)
    rw = jnp.stack([row_p.reshape(-1, 2 * _K),
                    ew_i.reshape(-1, 2 * _K)], axis=1).reshape(-1)

    acc1 = _sc_message(_pack_bf16(g1), rw, col_p)
    h1, g2 = _tc2(acc1[0, :_N], acc1[1, :_N], g1, dis,
                  b1.reshape(1, _H), W2)

    acc2 = _sc_message(_pack_bf16(g2), rw, col_p)
    h2 = _tc3(acc2[0, :_N], acc2[1, :_N], g2, dis, b2.reshape(1, _H))

    return jnp.concatenate([h1, h2], axis=-1)


# speedup vs baseline: 1.1491x; 1.0050x over previous
import functools

import jax
import jax.numpy as jnp
from jax import lax
from jax.experimental import pallas as pl
from jax.experimental.pallas import tpu as pltpu
from jax.experimental.pallas import tpu_sc as plsc

_N = 10000
_E = 320000
_D = 128
_H = 128

_NC = 2
_NS = 16
_NW = _NC * _NS

_K = 48
_NCHUNK = 224
_DK = 128
_DCH = 84
_EPT = _K * _NCHUNK
_EPAD = _EPT * _NW

_NP = 10112
_RPT = _NP // _NS
_ZROWS = 128

_DEGP = 10240
_DPT = _DEGP // _NS

_ROWBLK = 1000
_GRID = _N // _ROWBLK

_mesh = plsc.VectorSubcoreMesh(core_axis_name="c", subcore_axis_name="s")


@functools.partial(
    pl.kernel,
    out_type=jax.ShapeDtypeStruct((_NC, _DEGP), jnp.float32),
    mesh=_mesh,
    scratch_types=[
        pltpu.VMEM_SHARED((_DEGP,), jnp.float32),
        pltpu.VMEM((_DK,), jnp.int32),
        pltpu.VMEM((_DK,), jnp.int32),
        pltpu.VMEM((_DK,), jnp.float32),
        pltpu.VMEM((_DK,), jnp.float32),
        pltpu.VMEM((_DPT,), jnp.float32),
        pltpu.SemaphoreType.DMA,
        pltpu.SemaphoreType.DMA,
        pltpu.SemaphoreType.DMA,
    ],
)
def _sc_degree(c_hbm, ew_hbm, out_hbm, deg_sh, c0, c1, w0, w1, zbuf,
               isem0, isem1, ssem):
    cid = lax.axis_index("c")
    sid = lax.axis_index("s")
    wid = sid * _NC + cid
    c_v = (c0, c1)
    ew_v = (w0, w1)
    isem = (isem0, isem1)

    for i in range(_DPT // 16):
        zbuf[pl.ds(i * 16, 16)] = jnp.zeros((16,), jnp.float32)
    pltpu.sync_copy(zbuf, deg_sh.at[pl.ds(sid * _DPT, _DPT)])
    plsc.subcore_barrier()

    base = wid * _EPT

    def idx_descs(i, b):
        off = base + i * _DK
        return (
            pltpu.make_async_copy(c_hbm.at[pl.ds(off, _DK)], c_v[b],
                                  isem[b]),
            pltpu.make_async_copy(ew_hbm.at[pl.ds(off, _DK)], ew_v[b],
                                  isem[b]),
        )

    for d in idx_descs(0, 0):
        d.start()
    for d in idx_descs(0, 0):
        d.wait()

    def chunk2(io, carry):
        for b in range(2):
            i = io * 2 + b
            nb = 1 - b

            @pl.when(i + 1 < _DCH)
            def _next_idx():
                for d in idx_descs(i + 1, nb):
                    d.start()

            @pl.when(i > 0)
            def _drain():
                pltpu.make_async_copy(ew_v[nb], deg_sh.at[c_v[nb]],
                                      ssem).wait()

            pltpu.async_copy(ew_v[b], deg_sh.at[c_v[b]], ssem, add=True)

            @pl.when(i + 1 < _DCH)
            def _land_idx():
                for d in idx_descs(i + 1, nb):
                    d.wait()
        return carry

    lax.fori_loop(0, _DCH // 2, chunk2, 0)
    pltpu.make_async_copy(ew_v[1], deg_sh.at[c_v[1]], ssem).wait()
    plsc.subcore_barrier()
    pltpu.sync_copy(deg_sh.at[pl.ds(sid * _DPT, _DPT)],
                    out_hbm.at[cid, pl.ds(sid * _DPT, _DPT)])


@functools.partial(
    pl.kernel,
    out_type=jax.ShapeDtypeStruct((_NC, _NP, _H), jnp.float32),
    mesh=_mesh,
    compiler_params=pltpu.CompilerParams(use_tc_tiling_on_sc=False,
                                        needs_layout_passes=False),
    scratch_types=[
        pltpu.VMEM_SHARED((_NP, _H), jnp.float32),
        pltpu.VMEM_SHARED((_NP, _H // 2), jnp.int32),
        pltpu.VMEM((_K, _H // 2), jnp.int32),
        pltpu.VMEM((_K, _H), jnp.float32),
        pltpu.VMEM((4 * _K,), jnp.int32),
        pltpu.VMEM((4 * _K,), jnp.int32),
        pltpu.VMEM((_K,), jnp.int32),
        pltpu.VMEM((_K,), jnp.int32),
        pltpu.SemaphoreType.DMA,
        pltpu.SemaphoreType.DMA,
        pltpu.SemaphoreType.DMA,
        pltpu.SemaphoreType.DMA,
        pltpu.SemaphoreType.DMA,
        pltpu.SemaphoreType.DMA,
    ],
)
def _sc_message(gp_hbm, rw_hbm, c_hbm, out_hbm,
                acc, g_sh, gb, stage, rw0, rw1, cb0, cb1,
                gsem, ssem, rwsem0, rwsem1, csem0, csem1):
    cid = lax.axis_index("c")
    sid = lax.axis_index("s")
    wid = sid * _NC + cid
    rwb = (rw0, rw1)
    cb = (cb0, cb1)
    rwsem = (rwsem0, rwsem1)
    csem = (csem0, csem1)

    def zrow(i, carry):
        for q in range(_H // 16):
            stage[i, pl.ds(q * 16, 16)] = jnp.zeros((16,), jnp.float32)
        return carry

    lax.fori_loop(0, _K, zrow, 0)
    pltpu.sync_copy(gp_hbm.at[pl.ds(sid * _RPT, _RPT)],
                    g_sh.at[pl.ds(sid * _RPT, _RPT)])
    for t in range(13):
        pltpu.sync_copy(stage,
                        acc.at[pl.ds(sid * _RPT + t * _K, _K)])
    pltpu.sync_copy(stage.at[pl.ds(0, _RPT - 13 * _K)],
                    acc.at[pl.ds(sid * _RPT + 13 * _K, _RPT - 13 * _K)])
    plsc.subcore_barrier()

    nblk = _NCHUNK // 2
    rwbase = wid * nblk * 4 * _K
    cbase = wid * _EPT

    def rw_desc(blk, slot):
        return pltpu.make_async_copy(
            rw_hbm.at[pl.ds(rwbase + blk * 4 * _K, 4 * _K)], rwb[slot],
            rwsem[slot])

    def c_desc(i, slot):
        return pltpu.make_async_copy(
            c_hbm.at[pl.ds(cbase + i * _K, _K)], cb[slot], csem[slot])

    for d in (rw_desc(0, 0), c_desc(0, 0)):
        d.start()
    for d in (rw_desc(0, 0), c_desc(0, 0)):
        d.wait()
    pltpu.async_copy(g_sh.at[rw0.at[pl.ds(0, _K)]], gb, gsem)

    def blk2(bo, carry):
        for bslot in range(2):
            blk = bo * 2 + bslot
            for sub in range(2):
                i = blk * 2 + sub
                pltpu.make_async_copy(
                    g_sh.at[rwb[bslot].at[pl.ds(sub * _K, _K)]], gb,
                    gsem).wait()

                @pl.when(i > 0)
                def _drain_scatter():
                    pltpu.make_async_copy(
                        stage, acc.at[cb[1 - sub]], ssem).wait()

                if sub == 0:
                    @pl.when(blk + 1 < nblk)
                    def _next_rw():
                        rw_desc(blk + 1, 1 - bslot).start()

                @pl.when(i + 1 < _NCHUNK)
                def _next_c():
                    c_desc(i + 1, 1 - sub).start()

                def scale16(jo, inner):
                    j0 = jo * 16
                    wv = plsc.bitcast(
                        rwb[bslot][pl.ds(2 * _K + sub * _K + j0, 16)],
                        jnp.float32)
                    for jj in range(16):
                        w = lax.gather(
                            wv, jnp.full((16, 1), jj, jnp.int32),
                            lax.GatherDimensionNumbers(
                                offset_dims=(), collapsed_slice_dims=(0,),
                                start_index_map=(0,)),
                            (1,),
                            mode=lax.GatherScatterMode.PROMISE_IN_BOUNDS)
                        for q in range(_H // 32):
                            u = gb[j0 + jj, pl.ds(q * 16, 16)]
                            lo = plsc.bitcast(u << 16, jnp.float32)
                            hi = plsc.bitcast(
                                u & jnp.int32(-65536), jnp.float32)
                            stage[j0 + jj, pl.ds(q * 32, 16)] = lo * w
                            stage[j0 + jj, pl.ds(q * 32 + 16, 16)] = hi * w
                    return inner

                lax.fori_loop(0, _K // 16, scale16, 0)

                if sub == 0:
                    pltpu.async_copy(
                        g_sh.at[rwb[bslot].at[pl.ds(_K, _K)]], gb, gsem)
                else:
                    @pl.when(blk + 1 < nblk)
                    def _next_gather():
                        rw_desc(blk + 1, 1 - bslot).wait()
                        pltpu.async_copy(
                            g_sh.at[rwb[1 - bslot].at[pl.ds(0, _K)]], gb,
                            gsem)

                pltpu.async_copy(stage, acc.at[cb[sub]], ssem, add=True)

                @pl.when(i + 1 < _NCHUNK)
                def _land_c():
                    c_desc(i + 1, 1 - sub).wait()
        return carry

    lax.fori_loop(0, nblk // 2, blk2, 0)
    pltpu.make_async_copy(stage, acc.at[cb[1]], ssem).wait()
    plsc.subcore_barrier()
    for t in range(13):
        o = sid * _RPT + t * _K
        pltpu.sync_copy(acc.at[pl.ds(o, _K)], out_hbm.at[cid, pl.ds(o, _K)])
    o = sid * _RPT + 13 * _K
    pltpu.sync_copy(acc.at[pl.ds(o, _RPT - 13 * _K)],
                    out_hbm.at[cid, pl.ds(o, _RPT - 13 * _K)])


def _tc1a_body(x_ref, w1_ref, h_ref):
    h_ref[...] = jnp.dot(x_ref[...], w1_ref[...],
                         preferred_element_type=jnp.float32)


def _tc1b_body(d0_ref, d1_ref, h_ref, dis_ref, g1_ref):
    deg = d0_ref[...] + d1_ref[...] + 1.0
    dis = jnp.where(deg > 0.0, lax.rsqrt(deg), 0.0)
    dis_ref[...] = dis
    g1_ref[...] = h_ref[...] * dis


def _tc2_body(a0_ref, a1_ref, g1_ref, dis_ref, b1_ref, w2_ref,
              h1_ref, g2_ref):
    dis = dis_ref[...]
    pre = (a0_ref[...] + a1_ref[...] + g1_ref[...]) * dis + b1_ref[...]
    h1 = jnp.maximum(pre, 0.0)
    h1_ref[...] = h1
    g2_ref[...] = jnp.dot(h1, w2_ref[...],
                          preferred_element_type=jnp.float32) * dis


def _tc3_body(a0_ref, a1_ref, g2_ref, dis_ref, b2_ref, h1_ref, out_ref):
    pre = ((a0_ref[...] + a1_ref[...] + g2_ref[...]) * dis_ref[...]
           + b2_ref[...])
    out_ref[...] = jnp.concatenate(
        [h1_ref[...], jnp.maximum(pre, 0.0)], axis=-1)


def _row_blk(shape_cols):
    return pl.BlockSpec((_ROWBLK, shape_cols), lambda i: (i, 0))


def _full_blk(rows, cols):
    return pl.BlockSpec((rows, cols), lambda i: (0, 0))


_tc1a = pl.pallas_call(
    _tc1a_body,
    grid=(_GRID,),
    in_specs=[_row_blk(_D), _full_blk(_D, _H)],
    out_specs=_row_blk(_H),
    out_shape=jax.ShapeDtypeStruct((_N, _H), jnp.float32),
)

_tc1b = pl.pallas_call(
    _tc1b_body,
    grid=(_GRID,),
    in_specs=[_row_blk(1), _row_blk(1), _row_blk(_H)],
    out_specs=[_row_blk(1), _row_blk(_H)],
    out_shape=[
        jax.ShapeDtypeStruct((_N, 1), jnp.float32),
        jax.ShapeDtypeStruct((_N, _H), jnp.float32),
    ],
)

_tc2 = pl.pallas_call(
    _tc2_body,
    grid=(_GRID,),
    in_specs=[
        _row_blk(_H), _row_blk(_H), _row_blk(_H), _row_blk(1),
        _full_blk(1, _H), _full_blk(_H, _H),
    ],
    out_specs=[_row_blk(_H), _row_blk(_H)],
    out_shape=[
        jax.ShapeDtypeStruct((_N, _H), jnp.float32),
        jax.ShapeDtypeStruct((_N, _H), jnp.float32),
    ],
)

_tc3 = pl.pallas_call(
    _tc3_body,
    grid=(_GRID,),
    in_specs=[
        _row_blk(_H), _row_blk(_H), _row_blk(_H), _row_blk(1),
        _full_blk(1, _H), _row_blk(_H),
    ],
    out_specs=_row_blk(2 * _H),
    out_shape=jax.ShapeDtypeStruct((_N, 2 * _H), jnp.float32),
)


def _pack_bf16(g):
    ge = g.astype(jnp.bfloat16).reshape(_N, _H // 32, 2, 16)
    gt = ge.transpose(0, 1, 3, 2)
    gi = lax.bitcast_convert_type(gt, jnp.int32).reshape(_N, _H // 2)
    return jnp.concatenate(
        [gi, jnp.zeros((_NP - _N, _H // 2), jnp.int32)], axis=0)


@jax.jit
def kernel(x, edge_index, edge_weights, W1, b1, W2, b2):
    row = edge_index[0]
    col = edge_index[1]
    pad = _EPAD - _E
    row_p = jnp.concatenate([row, jnp.zeros((pad,), jnp.int32)])
    col_p = jnp.concatenate([col, jnp.zeros((pad,), jnp.int32)])
    ew_p = jnp.concatenate([edge_weights, jnp.zeros((pad,), jnp.float32)])

    deg_parts = _sc_degree(col_p, ew_p)
    d0 = deg_parts[0, :_N].reshape(_N, 1)
    d1 = deg_parts[1, :_N].reshape(_N, 1)

    hraw = _tc1a(x, W1)
    dis, g1 = _tc1b(d0, d1, hraw)

    ew_i = lax.bitcast_convert_type(ew_p, jnp.int32)
    rw = jnp.stack([row_p.reshape(-1, 2 * _K),
                    ew_i.reshape(-1, 2 * _K)], axis=1).reshape(-1)

    acc1 = _sc_message(_pack_bf16(g1), rw, col_p)
    h1, g2 = _tc2(acc1[0, :_N], acc1[1, :_N], g1, dis,
                  b1.reshape(1, _H), W2)

    acc2 = _sc_message(_pack_bf16(g2), rw, col_p)
    return _tc3(acc2[0, :_N], acc2[1, :_N], g2, dis,
                b2.reshape(1, _H), h1)


# speedup vs baseline: 1.9230x; 1.6735x over previous
import functools

import jax
import jax.numpy as jnp
from jax import lax
from jax.experimental import pallas as pl
from jax.experimental.pallas import tpu as pltpu
from jax.experimental.pallas import tpu_sc as plsc

_N = 10000
_E = 320000
_D = 128
_H = 128

_NC = 2
_NS = 16
_NW = _NC * _NS

_K = 48
_NCHUNK = 224
_DK = 128
_DCH = 84
_EPT = _K * _NCHUNK
_EPAD = _EPT * _NW

_NP = 10112
_RPT = _NP // _NS
_ZROWS = 128

_DEGP = 10240
_DPT = _DEGP // _NS

_ROWBLK = 1000
_GRID = _N // _ROWBLK

_mesh = plsc.VectorSubcoreMesh(core_axis_name="c", subcore_axis_name="s")


@functools.partial(
    pl.kernel,
    out_type=jax.ShapeDtypeStruct((_NC, _DEGP), jnp.float32),
    mesh=_mesh,
    scratch_types=[
        pltpu.VMEM_SHARED((_DEGP,), jnp.float32),
        pltpu.VMEM((_DK,), jnp.int32),
        pltpu.VMEM((_DK,), jnp.int32),
        pltpu.VMEM((_DK,), jnp.float32),
        pltpu.VMEM((_DK,), jnp.float32),
        pltpu.VMEM((_DPT,), jnp.float32),
        pltpu.SemaphoreType.DMA,
        pltpu.SemaphoreType.DMA,
        pltpu.SemaphoreType.DMA,
    ],
)
def _sc_degree(c_hbm, ew_hbm, out_hbm, deg_sh, c0, c1, w0, w1, zbuf,
               isem0, isem1, ssem):
    cid = lax.axis_index("c")
    sid = lax.axis_index("s")
    wid = sid * _NC + cid
    c_v = (c0, c1)
    ew_v = (w0, w1)
    isem = (isem0, isem1)

    for i in range(_DPT // 16):
        zbuf[pl.ds(i * 16, 16)] = jnp.zeros((16,), jnp.float32)
    pltpu.sync_copy(zbuf, deg_sh.at[pl.ds(sid * _DPT, _DPT)])
    plsc.subcore_barrier()

    base = wid * _EPT

    def idx_descs(i, b):
        off = base + i * _DK
        return (
            pltpu.make_async_copy(c_hbm.at[pl.ds(off, _DK)], c_v[b],
                                  isem[b]),
            pltpu.make_async_copy(ew_hbm.at[pl.ds(off, _DK)], ew_v[b],
                                  isem[b]),
        )

    for d in idx_descs(0, 0):
        d.start()
    for d in idx_descs(0, 0):
        d.wait()

    def chunk2(io, carry):
        for b in range(2):
            i = io * 2 + b
            nb = 1 - b

            @pl.when(i + 1 < _DCH)
            def _next_idx():
                for d in idx_descs(i + 1, nb):
                    d.start()

            @pl.when(i > 0)
            def _drain():
                pltpu.make_async_copy(ew_v[nb], deg_sh.at[c_v[nb]],
                                      ssem).wait()

            pltpu.async_copy(ew_v[b], deg_sh.at[c_v[b]], ssem, add=True)

            @pl.when(i + 1 < _DCH)
            def _land_idx():
                for d in idx_descs(i + 1, nb):
                    d.wait()
        return carry

    lax.fori_loop(0, _DCH // 2, chunk2, 0)
    pltpu.make_async_copy(ew_v[1], deg_sh.at[c_v[1]], ssem).wait()
    plsc.subcore_barrier()
    pltpu.sync_copy(deg_sh.at[pl.ds(sid * _DPT, _DPT)],
                    out_hbm.at[cid, pl.ds(sid * _DPT, _DPT)])


@functools.partial(
    pl.kernel,
    out_type=jax.ShapeDtypeStruct((_NC, _NP, _H), jnp.float32),
    mesh=_mesh,
    compiler_params=pltpu.CompilerParams(use_tc_tiling_on_sc=False,
                                        needs_layout_passes=False),
    scratch_types=[
        pltpu.VMEM_SHARED((_NP, _H), jnp.float32),
        pltpu.VMEM_SHARED((_NP, _H // 2), jnp.int32),
        pltpu.VMEM((_K, _H // 2), jnp.int32),
        pltpu.VMEM((_K, _H), jnp.float32),
        pltpu.VMEM((4 * _K,), jnp.int32),
        pltpu.VMEM((4 * _K,), jnp.int32),
        pltpu.VMEM((_K,), jnp.int32),
        pltpu.VMEM((_K,), jnp.int32),
        pltpu.SemaphoreType.DMA,
        pltpu.SemaphoreType.DMA,
        pltpu.SemaphoreType.DMA,
        pltpu.SemaphoreType.DMA,
        pltpu.SemaphoreType.DMA,
        pltpu.SemaphoreType.DMA,
    ],
)
def _sc_message(gp_hbm, rw_hbm, c_hbm, out_hbm,
                acc, g_sh, gb, stage, rw0, rw1, cb0, cb1,
                gsem, ssem, rwsem0, rwsem1, csem0, csem1):
    cid = lax.axis_index("c")
    sid = lax.axis_index("s")
    wid = sid * _NC + cid
    rwb = (rw0, rw1)
    cb = (cb0, cb1)
    rwsem = (rwsem0, rwsem1)
    csem = (csem0, csem1)

    def zrow(i, carry):
        for q in range(_H // 16):
            stage[i, pl.ds(q * 16, 16)] = jnp.zeros((16,), jnp.float32)
        return carry

    lax.fori_loop(0, _K, zrow, 0)
    pltpu.sync_copy(gp_hbm.at[pl.ds(sid * _RPT, _RPT)],
                    g_sh.at[pl.ds(sid * _RPT, _RPT)])
    for t in range(13):
        pltpu.sync_copy(stage,
                        acc.at[pl.ds(sid * _RPT + t * _K, _K)])
    pltpu.sync_copy(stage.at[pl.ds(0, _RPT - 13 * _K)],
                    acc.at[pl.ds(sid * _RPT + 13 * _K, _RPT - 13 * _K)])
    plsc.subcore_barrier()

    nblk = _NCHUNK // 2
    rwbase = wid * nblk * 4 * _K
    cbase = wid * _EPT

    def rw_desc(blk, slot):
        return pltpu.make_async_copy(
            rw_hbm.at[pl.ds(rwbase + blk * 4 * _K, 4 * _K)], rwb[slot],
            rwsem[slot])

    def c_desc(i, slot):
        return pltpu.make_async_copy(
            c_hbm.at[pl.ds(cbase + i * _K, _K)], cb[slot], csem[slot])

    for d in (rw_desc(0, 0), c_desc(0, 0)):
        d.start()
    for d in (rw_desc(0, 0), c_desc(0, 0)):
        d.wait()
    pltpu.async_copy(g_sh.at[rw0.at[pl.ds(0, _K)]], gb, gsem)

    def blk2(bo, carry):
        for bslot in range(2):
            blk = bo * 2 + bslot
            for sub in range(2):
                i = blk * 2 + sub
                pltpu.make_async_copy(
                    g_sh.at[rwb[bslot].at[pl.ds(sub * _K, _K)]], gb,
                    gsem).wait()

                @pl.when(i > 0)
                def _drain_scatter():
                    pltpu.make_async_copy(
                        stage, acc.at[cb[1 - sub]], ssem).wait()

                if sub == 0:
                    @pl.when(blk + 1 < nblk)
                    def _next_rw():
                        rw_desc(blk + 1, 1 - bslot).start()

                @pl.when(i + 1 < _NCHUNK)
                def _next_c():
                    c_desc(i + 1, 1 - sub).start()

                def scale16(jo, inner):
                    j0 = jo * 16
                    wv = plsc.bitcast(
                        rwb[bslot][pl.ds(2 * _K + sub * _K + j0, 16)],
                        jnp.float32)
                    for jj in range(16):
                        w = lax.gather(
                            wv, jnp.full((16, 1), jj, jnp.int32),
                            lax.GatherDimensionNumbers(
                                offset_dims=(), collapsed_slice_dims=(0,),
                                start_index_map=(0,)),
                            (1,),
                            mode=lax.GatherScatterMode.PROMISE_IN_BOUNDS)
                        for q in range(_H // 32):
                            u = gb[j0 + jj, pl.ds(q * 16, 16)]
                            lo = plsc.bitcast(u << 16, jnp.float32)
                            hi = plsc.bitcast(
                                u & jnp.int32(-65536), jnp.float32)
                            stage[j0 + jj, pl.ds(q * 32, 16)] = lo * w
                            stage[j0 + jj, pl.ds(q * 32 + 16, 16)] = hi * w
                    return inner

                for _jo in range(_K // 16):
                    scale16(_jo, 0)

                if sub == 0:
                    pltpu.async_copy(
                        g_sh.at[rwb[bslot].at[pl.ds(_K, _K)]], gb, gsem)
                else:
                    @pl.when(blk + 1 < nblk)
                    def _next_gather():
                        rw_desc(blk + 1, 1 - bslot).wait()
                        pltpu.async_copy(
                            g_sh.at[rwb[1 - bslot].at[pl.ds(0, _K)]], gb,
                            gsem)

                pltpu.async_copy(stage, acc.at[cb[sub]], ssem, add=True)

                @pl.when(i + 1 < _NCHUNK)
                def _land_c():
                    c_desc(i + 1, 1 - sub).wait()
        return carry

    lax.fori_loop(0, nblk // 2, blk2, 0)
    pltpu.make_async_copy(stage, acc.at[cb[1]], ssem).wait()
    plsc.subcore_barrier()
    for t in range(13):
        o = sid * _RPT + t * _K
        pltpu.sync_copy(acc.at[pl.ds(o, _K)], out_hbm.at[cid, pl.ds(o, _K)])
    o = sid * _RPT + 13 * _K
    pltpu.sync_copy(acc.at[pl.ds(o, _RPT - 13 * _K)],
                    out_hbm.at[cid, pl.ds(o, _RPT - 13 * _K)])


def _tc1a_body(x_ref, w1_ref, h_ref):
    h_ref[...] = jnp.dot(x_ref[...], w1_ref[...],
                         preferred_element_type=jnp.float32)


def _tc1b_body(d0_ref, d1_ref, h_ref, dis_ref, g1_ref):
    deg = d0_ref[...] + d1_ref[...] + 1.0
    dis = jnp.where(deg > 0.0, lax.rsqrt(deg), 0.0)
    dis_ref[...] = dis
    g1_ref[...] = h_ref[...] * dis


def _tc2_body(a0_ref, a1_ref, g1_ref, dis_ref, b1_ref, w2_ref,
              h1_ref, g2_ref):
    dis = dis_ref[...]
    pre = (a0_ref[...] + a1_ref[...] + g1_ref[...]) * dis + b1_ref[...]
    h1 = jnp.maximum(pre, 0.0)
    h1_ref[...] = h1
    g2_ref[...] = jnp.dot(h1, w2_ref[...],
                          preferred_element_type=jnp.float32) * dis


def _tc3_body(a0_ref, a1_ref, g2_ref, dis_ref, b2_ref, h1_ref, out_ref):
    pre = ((a0_ref[...] + a1_ref[...] + g2_ref[...]) * dis_ref[...]
           + b2_ref[...])
    out_ref[...] = jnp.concatenate(
        [h1_ref[...], jnp.maximum(pre, 0.0)], axis=-1)


def _row_blk(shape_cols):
    return pl.BlockSpec((_ROWBLK, shape_cols), lambda i: (i, 0))


def _full_blk(rows, cols):
    return pl.BlockSpec((rows, cols), lambda i: (0, 0))


_tc1a = pl.pallas_call(
    _tc1a_body,
    grid=(_GRID,),
    in_specs=[_row_blk(_D), _full_blk(_D, _H)],
    out_specs=_row_blk(_H),
    out_shape=jax.ShapeDtypeStruct((_N, _H), jnp.float32),
)

_tc1b = pl.pallas_call(
    _tc1b_body,
    grid=(_GRID,),
    in_specs=[_row_blk(1), _row_blk(1), _row_blk(_H)],
    out_specs=[_row_blk(1), _row_blk(_H)],
    out_shape=[
        jax.ShapeDtypeStruct((_N, 1), jnp.float32),
        jax.ShapeDtypeStruct((_N, _H), jnp.float32),
    ],
)

_tc2 = pl.pallas_call(
    _tc2_body,
    grid=(_GRID,),
    in_specs=[
        _row_blk(_H), _row_blk(_H), _row_blk(_H), _row_blk(1),
        _full_blk(1, _H), _full_blk(_H, _H),
    ],
    out_specs=[_row_blk(_H), _row_blk(_H)],
    out_shape=[
        jax.ShapeDtypeStruct((_N, _H), jnp.float32),
        jax.ShapeDtypeStruct((_N, _H), jnp.float32),
    ],
)

_tc3 = pl.pallas_call(
    _tc3_body,
    grid=(_GRID,),
    in_specs=[
        _row_blk(_H), _row_blk(_H), _row_blk(_H), _row_blk(1),
        _full_blk(1, _H), _row_blk(_H),
    ],
    out_specs=_row_blk(2 * _H),
    out_shape=jax.ShapeDtypeStruct((_N, 2 * _H), jnp.float32),
)


def _pack_bf16(g):
    ge = g.astype(jnp.bfloat16).reshape(_N, _H // 32, 2, 16)
    gt = ge.transpose(0, 1, 3, 2)
    gi = lax.bitcast_convert_type(gt, jnp.int32).reshape(_N, _H // 2)
    return jnp.concatenate(
        [gi, jnp.zeros((_NP - _N, _H // 2), jnp.int32)], axis=0)


@jax.jit
def kernel(x, edge_index, edge_weights, W1, b1, W2, b2):
    row = edge_index[0]
    col = edge_index[1]
    pad = _EPAD - _E
    row_p = jnp.concatenate([row, jnp.zeros((pad,), jnp.int32)])
    col_p = jnp.concatenate([col, jnp.zeros((pad,), jnp.int32)])
    ew_p = jnp.concatenate([edge_weights, jnp.zeros((pad,), jnp.float32)])

    deg_parts = _sc_degree(col_p, ew_p)
    d0 = deg_parts[0, :_N].reshape(_N, 1)
    d1 = deg_parts[1, :_N].reshape(_N, 1)

    hraw = _tc1a(x, W1)
    dis, g1 = _tc1b(d0, d1, hraw)

    ew_i = lax.bitcast_convert_type(ew_p, jnp.int32)
    rw = jnp.stack([row_p.reshape(-1, 2 * _K),
                    ew_i.reshape(-1, 2 * _K)], axis=1).reshape(-1)

    acc1 = _sc_message(_pack_bf16(g1), rw, col_p)
    h1, g2 = _tc2(acc1[0, :_N], acc1[1, :_N], g1, dis,
                  b1.reshape(1, _H), W2)

    acc2 = _sc_message(_pack_bf16(g2), rw, col_p)
    return _tc3(acc2[0, :_N], acc2[1, :_N], g2, dis,
                b2.reshape(1, _H), h1)
